# baseline (speedup 1.0000x reference)
import functools

import jax
import jax.numpy as jnp
from jax import lax
from jax.experimental import pallas as pl
from jax.experimental.pallas import tpu as pltpu
from jax.experimental.pallas import tpu_sc as plsc

NC = 2
NS = 16
NW = NC * NS
N = 50000
E = 800000
NPAD = 50176
MPAD = 50176
H = 128
CW = 32
B = 128
TPB = 196
E2 = NW * TPB * B
EROWS = E2 // B
SB = 28
NSB = TPB // SB
RING = 4
NBLK = SB // RING
NPAIR = 24
RPT = NPAD // NS
RC = 224
NCOPY = RPT // RC
DUMP = N + 48


def _segsum_kernel(Ks):
    mesh = plsc.VectorSubcoreMesh(core_axis_name="c", subcore_axis_name="s")
    out_type = tuple(jax.ShapeDtypeStruct((NC * K * NPAD, CW), jnp.float32)
                     for K in Ks)
    scratch = dict(
        sidxA=pltpu.VMEM((RING, B), jnp.int32),
        didxA=pltpu.VMEM((RING, B), jnp.int32),
        sidxB=pltpu.VMEM((RING, B), jnp.int32),
        didxB=pltpu.VMEM((RING, B), jnp.int32),
        rows=pltpu.VMEM((RING, B, CW), jnp.float32),
        acc=pltpu.VMEM_SHARED((NPAD, CW), jnp.float32),
        sem1=pltpu.SemaphoreType.DMA,
        sem2=pltpu.SemaphoreType.DMA,
        sem3=pltpu.SemaphoreType.DMA,
    )

    def body(*refs, sidxA, didxA, sidxB, didxB, rows, acc,
             sem1, sem2, sem3):
        nconv = len(Ks)
        pos = 0
        convs = []
        for K in Ks:
            convs.append(refs[pos:pos + K])
            pos += K
        edges = []
        for _ in range(nconv):
            edges.append(refs[pos:pos + 2])
            pos += 2
        zf_h = refs[pos]
        parts = refs[pos + 1:pos + 1 + nconv]

        c = lax.axis_index("c")
        s = lax.axis_index("s")
        rbase = (c * NS + s) * TPB

        for ci in range(nconv):
            xcs = convs[ci]
            src_h, dst_h = edges[ci]
            part_h = parts[ci]
            K = Ks[ci]
            _conv(xcs, src_h, dst_h, part_h, K, c, s, rbase, zf_h,
                  sidxA, didxA, sidxB, didxB, rows, acc,
                  sem1, sem2, sem3)

    def _conv(xcs, src_h, dst_h, part_h, K, c, s, rbase, zf_h,
              sidxA, didxA, sidxB, didxB, rows, acc,
              sem1, sem2, sem3):
        def load_idx(row0, sdst, ddst):
            ha = pltpu.async_copy(src_h.at[pl.ds(row0, RING)], sdst, sem3)
            hb = pltpu.async_copy(dst_h.at[pl.ds(row0, RING)], ddst, sem3)
            return ha, hb

        for k in range(K):
            pltpu.sync_copy(zf_h, acc.at[pl.ds(s * RPT, RPT)])
            plsc.subcore_barrier()

            def process(sidx, didx):
                hs = []
                for r in range(RING):
                    hs.append(pltpu.async_copy(
                        xcs[k].at[sidx.at[r]], rows.at[r], sem1))
                scs = []
                for r in range(RING):
                    hs[r].wait()
                    scs.append(pltpu.async_copy(
                        rows.at[r], acc.at[didx.at[r]], sem2, add=True))
                for r in range(RING):
                    scs[r].wait()

            ha, hb = load_idx(rbase, sidxA, didxA)
            ha.wait()
            hb.wait()

            def pair(i2, carry):
                j0 = rbase + i2 * 2 * RING
                ha, hb = load_idx(j0 + RING, sidxB, didxB)
                process(sidxA, didxA)
                ha.wait()
                hb.wait()
                h2a, h2b = load_idx(j0 + 2 * RING, sidxA, didxA)
                process(sidxB, didxB)
                h2a.wait()
                h2b.wait()
                return carry

            lax.fori_loop(0, NPAIR, pair, 0)
            process(sidxA, didxA)
            plsc.subcore_barrier()

            base = (c * K + k) * NPAD
            for i in range(NCOPY):
                r0 = s * RPT + i * RC
                pltpu.sync_copy(acc.at[pl.ds(r0, RC)],
                                part_h.at[pl.ds(base + r0, RC)])
            plsc.subcore_barrier()

    return pl.kernel(body, out_type=out_type, mesh=mesh,
                     scratch_types=scratch,
                     compiler_params=pltpu.CompilerParams(
                         use_tc_tiling_on_sc=False,
                         needs_layout_passes=False))


def _segsum(convs, consts):
    Ks = [len(xc) for xc, _, _ in convs]
    fn = _segsum_kernel(Ks)
    zf, zf8, ones8 = consts
    args = []
    for xc, _, _ in convs:
        args.extend(xc)
    for _, s2, d2 in convs:
        args.extend([s2, d2])
    args.append(zf)
    outs = fn(*args)
    return [o.reshape(NC, K, NPAD, CW) for o, K in zip(outs, Ks)]


def _cnt_kernel():
    mesh = plsc.VectorSubcoreMesh(core_axis_name="c", subcore_axis_name="s")
    out_type = tuple(jax.ShapeDtypeStruct((NC * NPAD, 8), jnp.float32)
                     for _ in range(2))
    scratch = dict(
        didx2=pltpu.VMEM((SB, B), jnp.int32),
        ones_v=pltpu.VMEM((B, 8), jnp.float32),
        cnt_acc=pltpu.VMEM_SHARED((NPAD, 8), jnp.float32),
        sem1=pltpu.SemaphoreType.DMA,
    )

    def body(dsta_h, dstb_h, zf8_h, ones_h, cnta_h, cntb_h, *,
             didx2, ones_v, cnt_acc, sem1):
        c = lax.axis_index("c")
        s = lax.axis_index("s")
        rbase = (c * NS + s) * TPB
        pltpu.sync_copy(ones_h, ones_v)
        for dst_h, cnt_h in ((dsta_h, cnta_h), (dstb_h, cntb_h)):
            pltpu.sync_copy(zf8_h, cnt_acc.at[pl.ds(s * RPT, RPT)])
            plsc.subcore_barrier()

            def sblk(sb, carry):
                row0 = rbase + sb * SB
                pltpu.sync_copy(dst_h.at[pl.ds(row0, SB)], didx2)

                def blk(jb, carry2):
                    j0 = jb * RING
                    scs = []
                    for r in range(RING):
                        scs.append(pltpu.async_copy(
                            ones_v, cnt_acc.at[didx2.at[j0 + r]], sem1,
                            add=True))
                    for r in range(RING):
                        scs[r].wait()
                    return carry2

                lax.fori_loop(0, NBLK, blk, 0)
                return carry

            lax.fori_loop(0, NSB, sblk, 0)
            plsc.subcore_barrier()
            for i in range(NCOPY):
                r0 = s * RPT + i * RC
                pltpu.sync_copy(cnt_acc.at[pl.ds(r0, RC)],
                                cnt_h.at[pl.ds(c * NPAD + r0, RC)])
            plsc.subcore_barrier()

    return pl.kernel(body, out_type=out_type, mesh=mesh,
                     scratch_types=scratch,
                     compiler_params=pltpu.CompilerParams(
                         use_tc_tiling_on_sc=False,
                         needs_layout_passes=False))


def _cnt2(dsta, dstb, consts):
    zf, zf8, ones8 = consts
    ca, cb = _cnt_kernel()(dsta, dstb, zf8, ones8)
    return ca.reshape(NC, NPAD, 8), cb.reshape(NC, NPAD, 8)


def _embed_body(x1_ref, xu_ref, ea_ref, eg_ref, out_ref, c0, c1, c2, c3):
    x1 = x1_ref[...]
    age_oh = (x1[:, 0:1] == lax.broadcasted_iota(jnp.int32, (1, 9), 1))
    gen_oh = (x1[:, 1:2] == lax.broadcasted_iota(jnp.int32, (1, 3), 1))
    age = jnp.dot(age_oh.astype(jnp.float32), ea_ref[...],
                  preferred_element_type=jnp.float32,
                  precision=lax.Precision.HIGHEST)
    gen = jnp.dot(gen_oh.astype(jnp.float32), eg_ref[...],
                  preferred_element_type=jnp.float32,
                  precision=lax.Precision.HIGHEST)
    xu = xu_ref[...]
    out_ref[:, 0:32] = age
    out_ref[:, 32:64] = gen
    out_ref[:, 64:128] = xu
    c0[...] = age
    c1[...] = gen
    c2[...] = xu[:, 0:32]
    c3[...] = xu[:, 32:64]


def _embed(x1p, xup, emb_age, emb_gender):
    RB = 256
    grid = (NPAD // RB,)
    outs = pl.pallas_call(
        _embed_body,
        grid=grid,
        in_specs=[
            pl.BlockSpec((RB, 2), lambda i: (i, 0)),
            pl.BlockSpec((RB, 64), lambda i: (i, 0)),
            pl.BlockSpec((9, 32), lambda i: (0, 0)),
            pl.BlockSpec((3, 32), lambda i: (0, 0)),
        ],
        out_specs=[pl.BlockSpec((RB, 128), lambda i: (i, 0))] +
                  [pl.BlockSpec((RB, 32), lambda i: (i, 0))] * 4,
        out_shape=[jax.ShapeDtypeStruct((NPAD, 128), jnp.float32)] +
                  [jax.ShapeDtypeStruct((NPAD, 32), jnp.float32)] * 4,
    )(x1p, xup, emb_age, emb_gender)
    return outs[0], list(outs[1:])


def _dense_body(K, emit_chunks, head, part_ref, cnt_ref, xd_ref, wl_ref,
                bl_ref, wr_ref, *rest):
    if head:
        wh_ref = rest[0]
        rest = rest[1:]
    out_ref = rest[0]
    chunk_refs = rest[1:1 + H // CW] if emit_chunks else ()
    head_ref = rest[-1] if head else None

    cnt = cnt_ref[0, :, 0] + cnt_ref[1, :, 0]
    inv = 1.0 / jnp.maximum(cnt, 1.0)
    wl = wl_ref[...]
    acc = jnp.zeros((out_ref.shape[0], H), jnp.float32)
    for k in range(K):
        aggk = (part_ref[0, k] + part_ref[1, k]) * inv[:, None]
        acc = acc + jnp.dot(aggk, wl[k * CW:(k + 1) * CW, :],
                            preferred_element_type=jnp.float32,
                  precision=lax.Precision.HIGHEST)
    y = acc + bl_ref[...] + jnp.dot(xd_ref[...], wr_ref[...],
                                    preferred_element_type=jnp.float32,
                  precision=lax.Precision.HIGHEST)
    y = jnp.maximum(y, 0.0)
    out_ref[...] = y
    if emit_chunks:
        for k in range(H // CW):
            chunk_refs[k][...] = y[:, k * CW:(k + 1) * CW]
    if head:
        head_ref[...] = jnp.dot(y, wh_ref[...],
                                preferred_element_type=jnp.float32,
                  precision=lax.Precision.HIGHEST)


def _dense(part, cnt, xd, wl, bl, wr, emit_chunks, wh=None):
    K = part.shape[1]
    Dd = xd.shape[1]
    RB = 256
    head = wh is not None
    in_specs = [
        pl.BlockSpec((NC, K, RB, CW), lambda i: (0, 0, i, 0)),
        pl.BlockSpec((NC, RB, 8), lambda i: (0, i, 0)),
        pl.BlockSpec((RB, Dd), lambda i: (i, 0)),
        pl.BlockSpec((H, H), lambda i: (0, 0)),
        pl.BlockSpec((H,), lambda i: (0,)),
        pl.BlockSpec((Dd, H), lambda i: (0, 0)),
    ]
    args = [part, cnt, xd, wl, bl, wr]
    if head:
        in_specs.append(pl.BlockSpec((H, 1), lambda i: (0, 0)))
        args.append(wh)
    out_specs = [pl.BlockSpec((RB, H), lambda i: (i, 0))]
    out_shape = [jax.ShapeDtypeStruct((NPAD, H), jnp.float32)]
    if emit_chunks:
        out_specs += [pl.BlockSpec((RB, CW), lambda i: (i, 0))] * (H // CW)
        out_shape += [jax.ShapeDtypeStruct((NPAD, CW), jnp.float32)] * (H // CW)
    if head:
        out_specs.append(pl.BlockSpec((RB, 1), lambda i: (i, 0)))
        out_shape.append(jax.ShapeDtypeStruct((NPAD, 1), jnp.float32))
    outs = pl.pallas_call(
        functools.partial(_dense_body, K, emit_chunks, head),
        grid=(NPAD // RB,),
        in_specs=in_specs,
        out_specs=out_specs,
        out_shape=out_shape,
    )(*args)
    full = outs[0]
    chunks = list(outs[1:1 + H // CW]) if emit_chunks else None
    hy = outs[-1] if head else None
    return full, chunks, hy


def _head_kernel():
    mesh = plsc.VectorSubcoreMesh(core_axis_name="c", subcore_axis_name="s")
    PW = MPAD // NW
    scratch = dict(
        yu_v=pltpu.VMEM((NPAD,), jnp.float32),
        ys_v=pltpu.VMEM((NPAD,), jnp.float32),
        mu_v=pltpu.VMEM((PW,), jnp.int32),
        ms_v=pltpu.VMEM((PW,), jnp.int32),
        b_v=pltpu.VMEM((16,), jnp.float32),
        out_v=pltpu.VMEM((PW,), jnp.float32),
    )

    def body(yu_h, ys_h, mu_h, ms_h, b_h, out_h, *, yu_v, ys_v, mu_v, ms_v,
             b_v, out_v):
        c = lax.axis_index("c")
        s = lax.axis_index("s")
        w = s * NC + c
        base = w * PW
        pltpu.sync_copy(yu_h, yu_v)
        pltpu.sync_copy(ys_h, ys_v)
        pltpu.sync_copy(mu_h.at[pl.ds(base, PW)], mu_v)
        pltpu.sync_copy(ms_h.at[pl.ds(base, PW)], ms_v)
        pltpu.sync_copy(b_h, b_v)
        bvec = b_v[...]

        def step(i, carry):
            iu = mu_v[pl.ds(i * 16, 16)]
            is_ = ms_v[pl.ds(i * 16, 16)]
            u = plsc.load_gather(yu_v, [iu])
            v = plsc.load_gather(ys_v, [is_])
            z = u + v + bvec
            out_v[pl.ds(i * 16, 16)] = 1.0 / (1.0 + jnp.exp(-z))
            return carry

        lax.fori_loop(0, PW // 16, step, 0)
        pltpu.sync_copy(out_v, out_h.at[pl.ds(base, PW)])

    return pl.kernel(
        body,
        out_type=jax.ShapeDtypeStruct((MPAD,), jnp.float32),
        mesh=mesh,
        scratch_types=scratch,
        compiler_params=pltpu.CompilerParams(use_tc_tiling_on_sc=False,
                                             needs_layout_passes=False),
    )


def _pad_rows(x, npad):
    return jnp.pad(x, ((0, npad - x.shape[0]),) + ((0, 0),) * (x.ndim - 1))


def kernel(x_user, x_seller, x1, edge_ub, edge_bu, mask,
           emb_age, emb_gender,
           Wl_us1, bl_us1, Wr_us1, Wl_su1, bl_su1, Wr_su1,
           Wl_us2, bl_us2, Wr_us2, Wl_su2, bl_su2, Wr_su2,
           W_lin, b_lin):
    zf = jnp.zeros((RPT, CW), jnp.float32)
    zf8 = jnp.zeros((RPT, 8), jnp.float32)
    ones8 = jnp.ones((B, 8), jnp.float32)
    consts = (zf, zf8, ones8)

    def pad_edges(ei):
        srcp = jnp.concatenate(
            [ei[0], jnp.zeros((E2 - E,), ei.dtype)]).reshape(EROWS, B)
        dstp = jnp.concatenate(
            [ei[1], jnp.full((E2 - E,), DUMP, ei.dtype)]).reshape(EROWS, B)
        return srcp, dstp

    x1p = _pad_rows(x1, NPAD)
    xup = _pad_rows(x_user, NPAD)
    xsp = _pad_rows(x_seller, NPAD)

    src_ub, dst_ub = pad_edges(edge_ub)
    src_bu, dst_bu = pad_edges(edge_bu)

    xu_full, xu_chunks, _ = (lambda o: (o[0], o[1], None))(_embed(
        x1p, xup, emb_age, emb_gender))

    xs_chunks = [xsp[:, 0:32], xsp[:, 32:64]]

    def chain(tok, c):
        zf_, zf8_, ones8_ = c
        t = tok * 1e-30
        return (zf_ + t, zf8_ + t, ones8_)

    cnt_ub, cnt_bu = _cnt2(dst_ub, dst_bu, consts)
    part_us1 = _segsum([(xu_chunks, src_ub, dst_ub)],
                       chain(cnt_bu[0, 0, 0], consts))[0]
    part_su1 = _segsum([(xs_chunks, src_bu, dst_bu)],
                       chain(part_us1[0, 0, 0, 0], consts))[0]

    xs1_full, xs1_chunks, _ = _dense(part_us1, cnt_ub, xsp, Wl_us1, bl_us1,
                                     Wr_us1, True)
    xu1_full, xu1_chunks, _ = _dense(part_su1, cnt_bu, xu_full, Wl_su1,
                                     bl_su1, Wr_su1, True)

    part_us2 = _segsum([(xu1_chunks, src_ub, dst_ub)],
                       chain(part_su1[0, 0, 0, 0], consts))[0]
    part_su2 = _segsum([(xs1_chunks, src_bu, dst_bu)],
                       chain(part_us2[0, 0, 0, 0], consts))[0]

    w_top = W_lin[0:128, :]
    w_bot = W_lin[128:256, :]
    _, _, ys = _dense(part_us2, cnt_ub, xs1_full, Wl_us2, bl_us2, Wr_us2,
                      False, wh=w_bot)
    _, _, yu = _dense(part_su2, cnt_bu, xu1_full, Wl_su2, bl_su2, Wr_su2,
                      False, wh=w_top)

    mu = jnp.pad(mask[:, 0], (0, MPAD - M_REAL))
    ms = jnp.pad(mask[:, 1], (0, MPAD - M_REAL))
    b16 = jnp.full((16,), b_lin[0], jnp.float32)

    out = _head_kernel()(yu.reshape(-1), ys.reshape(-1), mu, ms, b16)
    return out[:M_REAL]


M_REAL = 50000


# speedup vs baseline: 1.0277x; 1.0277x over previous
import functools

import jax
import jax.numpy as jnp
from jax import lax
from jax.experimental import pallas as pl
from jax.experimental.pallas import tpu as pltpu
from jax.experimental.pallas import tpu_sc as plsc

NC = 2
NS = 16
NW = NC * NS
N = 50000
E = 800000
NPAD = 50176
MPAD = 50176
H = 128
CW = 32
B = 128
TPB = 196
E2 = NW * TPB * B
EROWS = E2 // B
SB = 28
NSB = TPB // SB
RING = 4
NBLK = SB // RING
NPAIR = 24
RPT = NPAD // NS
RC = 224
NCOPY = RPT // RC
DUMP = N + 48


def _segsum_kernel(Ks):
    mesh = plsc.VectorSubcoreMesh(core_axis_name="c", subcore_axis_name="s")
    out_type = tuple(jax.ShapeDtypeStruct((NC * K * NPAD, CW), jnp.float32)
                     for K in Ks)
    scratch = dict(
        sidxA=pltpu.VMEM((RING, B), jnp.int32),
        didxA=pltpu.VMEM((RING, B), jnp.int32),
        sidxB=pltpu.VMEM((RING, B), jnp.int32),
        didxB=pltpu.VMEM((RING, B), jnp.int32),
        rows=pltpu.VMEM((RING, B, CW), jnp.float32),
        zbuf=pltpu.VMEM((RC, CW), jnp.float32),
        acc=pltpu.VMEM_SHARED((NPAD, CW), jnp.float32),
        sem1=pltpu.SemaphoreType.DMA,
        sem2=pltpu.SemaphoreType.DMA,
        sem3=pltpu.SemaphoreType.DMA,
    )

    def body(*refs, sidxA, didxA, sidxB, didxB, rows, zbuf, acc,
             sem1, sem2, sem3):
        nconv = len(Ks)
        pos = 0
        convs = []
        for K in Ks:
            convs.append(refs[pos:pos + K])
            pos += K
        edges = []
        for _ in range(nconv):
            edges.append(refs[pos:pos + 2])
            pos += 2
        zf_h = refs[pos]
        parts = refs[pos + 1:pos + 1 + nconv]

        c = lax.axis_index("c")
        s = lax.axis_index("s")
        rbase = (c * NS + s) * TPB

        for ci in range(nconv):
            xcs = convs[ci]
            src_h, dst_h = edges[ci]
            part_h = parts[ci]
            K = Ks[ci]
            _conv(xcs, src_h, dst_h, part_h, K, c, s, rbase, zf_h,
                  sidxA, didxA, sidxB, didxB, rows, zbuf, acc,
                  sem1, sem2, sem3)

    def _conv(xcs, src_h, dst_h, part_h, K, c, s, rbase, zf_h,
              sidxA, didxA, sidxB, didxB, rows, zbuf, acc,
              sem1, sem2, sem3):
        def load_idx(row0, sdst, ddst):
            ha = pltpu.async_copy(src_h.at[pl.ds(row0, RING)], sdst, sem3)
            hb = pltpu.async_copy(dst_h.at[pl.ds(row0, RING)], ddst, sem3)
            return ha, hb

        pltpu.sync_copy(zf_h, zbuf)
        for k in range(K):
            for i in range(NCOPY):
                r0 = s * RPT + i * RC
                pltpu.sync_copy(zbuf, acc.at[pl.ds(r0, RC)])
            plsc.subcore_barrier()

            def process(sidx, didx):
                hs = []
                for r in range(RING):
                    hs.append(pltpu.async_copy(
                        xcs[k].at[sidx.at[r]], rows.at[r], sem1))
                scs = []
                for r in range(RING):
                    hs[r].wait()
                    scs.append(pltpu.async_copy(
                        rows.at[r], acc.at[didx.at[r]], sem2, add=True))
                for r in range(RING):
                    scs[r].wait()

            ha, hb = load_idx(rbase, sidxA, didxA)
            ha.wait()
            hb.wait()

            def pair(i2, carry):
                j0 = rbase + i2 * 2 * RING
                ha, hb = load_idx(j0 + RING, sidxB, didxB)
                process(sidxA, didxA)
                ha.wait()
                hb.wait()
                h2a, h2b = load_idx(j0 + 2 * RING, sidxA, didxA)
                process(sidxB, didxB)
                h2a.wait()
                h2b.wait()
                return carry

            lax.fori_loop(0, NPAIR, pair, 0)
            process(sidxA, didxA)
            plsc.subcore_barrier()

            base = (c * K + k) * NPAD + s * RPT
            pltpu.sync_copy(acc.at[pl.ds(s * RPT, RPT)],
                            part_h.at[pl.ds(base, RPT)])
            plsc.subcore_barrier()

    return pl.kernel(body, out_type=out_type, mesh=mesh,
                     scratch_types=scratch,
                     compiler_params=pltpu.CompilerParams(
                         use_tc_tiling_on_sc=False,
                         needs_layout_passes=False))


def _segsum(convs, consts):
    Ks = [len(xc) for xc, _, _ in convs]
    fn = _segsum_kernel(Ks)
    zf, zf8, ones8 = consts
    args = []
    for xc, _, _ in convs:
        args.extend(xc)
    for _, s2, d2 in convs:
        args.extend([s2, d2])
    args.append(zf)
    outs = fn(*args)
    return [o.reshape(NC, K, NPAD, CW) for o, K in zip(outs, Ks)]


def _cnt_kernel():
    mesh = plsc.VectorSubcoreMesh(core_axis_name="c", subcore_axis_name="s")
    out_type = tuple(jax.ShapeDtypeStruct((NC * NPAD, 8), jnp.float32)
                     for _ in range(2))
    scratch = dict(
        didx2=pltpu.VMEM((SB, B), jnp.int32),
        ones_v=pltpu.VMEM((B, 8), jnp.float32),
        cnt_acc=pltpu.VMEM_SHARED((NPAD, 8), jnp.float32),
        sem1=pltpu.SemaphoreType.DMA,
    )

    def body(dsta_h, dstb_h, zf8_h, ones_h, cnta_h, cntb_h, *,
             didx2, ones_v, cnt_acc, sem1):
        c = lax.axis_index("c")
        s = lax.axis_index("s")
        rbase = (c * NS + s) * TPB
        pltpu.sync_copy(ones_h, ones_v)
        for dst_h, cnt_h in ((dsta_h, cnta_h), (dstb_h, cntb_h)):
            pltpu.sync_copy(zf8_h, cnt_acc.at[pl.ds(s * RPT, RPT)])
            plsc.subcore_barrier()

            def sblk(sb, carry):
                row0 = rbase + sb * SB
                pltpu.sync_copy(dst_h.at[pl.ds(row0, SB)], didx2)

                def blk(jb, carry2):
                    j0 = jb * RING
                    scs = []
                    for r in range(RING):
                        scs.append(pltpu.async_copy(
                            ones_v, cnt_acc.at[didx2.at[j0 + r]], sem1,
                            add=True))
                    for r in range(RING):
                        scs[r].wait()
                    return carry2

                lax.fori_loop(0, NBLK, blk, 0)
                return carry

            lax.fori_loop(0, NSB, sblk, 0)
            plsc.subcore_barrier()
            pltpu.sync_copy(cnt_acc.at[pl.ds(s * RPT, RPT)],
                            cnt_h.at[pl.ds(c * NPAD + s * RPT, RPT)])
            plsc.subcore_barrier()

    return pl.kernel(body, out_type=out_type, mesh=mesh,
                     scratch_types=scratch,
                     compiler_params=pltpu.CompilerParams(
                         use_tc_tiling_on_sc=False,
                         needs_layout_passes=False))


def _cnt2(dsta, dstb, consts):
    zf, zf8, ones8 = consts
    ca, cb = _cnt_kernel()(dsta, dstb, zf8, ones8)
    return ca.reshape(NC, NPAD, 8), cb.reshape(NC, NPAD, 8)


def _embed_body(x1_ref, xu_ref, ea_ref, eg_ref, out_ref, c0, c1, c2, c3):
    x1 = x1_ref[...]
    age_oh = (x1[:, 0:1] == lax.broadcasted_iota(jnp.int32, (1, 9), 1))
    gen_oh = (x1[:, 1:2] == lax.broadcasted_iota(jnp.int32, (1, 3), 1))
    age = jnp.dot(age_oh.astype(jnp.float32), ea_ref[...],
                  preferred_element_type=jnp.float32,
                  precision=lax.Precision.HIGHEST)
    gen = jnp.dot(gen_oh.astype(jnp.float32), eg_ref[...],
                  preferred_element_type=jnp.float32,
                  precision=lax.Precision.HIGHEST)
    xu = xu_ref[...]
    out_ref[:, 0:32] = age
    out_ref[:, 32:64] = gen
    out_ref[:, 64:128] = xu
    c0[...] = age
    c1[...] = gen
    c2[...] = xu[:, 0:32]
    c3[...] = xu[:, 32:64]


def _embed(x1p, xup, emb_age, emb_gender):
    RB = 256
    grid = (NPAD // RB,)
    outs = pl.pallas_call(
        _embed_body,
        grid=grid,
        in_specs=[
            pl.BlockSpec((RB, 2), lambda i: (i, 0)),
            pl.BlockSpec((RB, 64), lambda i: (i, 0)),
            pl.BlockSpec((9, 32), lambda i: (0, 0)),
            pl.BlockSpec((3, 32), lambda i: (0, 0)),
        ],
        out_specs=[pl.BlockSpec((RB, 128), lambda i: (i, 0))] +
                  [pl.BlockSpec((RB, 32), lambda i: (i, 0))] * 4,
        out_shape=[jax.ShapeDtypeStruct((NPAD, 128), jnp.float32)] +
                  [jax.ShapeDtypeStruct((NPAD, 32), jnp.float32)] * 4,
    )(x1p, xup, emb_age, emb_gender)
    return outs[0], list(outs[1:])


def _dense_body(K, emit_chunks, head, part_ref, cnt_ref, xd_ref, wl_ref,
                bl_ref, wr_ref, *rest):
    if head:
        wh_ref = rest[0]
        rest = rest[1:]
    out_ref = rest[0]
    chunk_refs = rest[1:1 + H // CW] if emit_chunks else ()
    head_ref = rest[-1] if head else None

    cnt = cnt_ref[0, :, 0] + cnt_ref[1, :, 0]
    inv = 1.0 / jnp.maximum(cnt, 1.0)
    wl = wl_ref[...]
    acc = jnp.zeros((out_ref.shape[0], H), jnp.float32)
    for k in range(K):
        aggk = (part_ref[0, k] + part_ref[1, k]) * inv[:, None]
        acc = acc + jnp.dot(aggk, wl[k * CW:(k + 1) * CW, :],
                            preferred_element_type=jnp.float32,
                  precision=lax.Precision.HIGHEST)
    y = acc + bl_ref[...] + jnp.dot(xd_ref[...], wr_ref[...],
                                    preferred_element_type=jnp.float32,
                  precision=lax.Precision.HIGHEST)
    y = jnp.maximum(y, 0.0)
    out_ref[...] = y
    if emit_chunks:
        for k in range(H // CW):
            chunk_refs[k][...] = y[:, k * CW:(k + 1) * CW]
    if head:
        head_ref[...] = jnp.dot(y, wh_ref[...],
                                preferred_element_type=jnp.float32,
                  precision=lax.Precision.HIGHEST)


def _dense(part, cnt, xd, wl, bl, wr, emit_chunks, wh=None):
    K = part.shape[1]
    Dd = xd.shape[1]
    RB = 256
    head = wh is not None
    in_specs = [
        pl.BlockSpec((NC, K, RB, CW), lambda i: (0, 0, i, 0)),
        pl.BlockSpec((NC, RB, 8), lambda i: (0, i, 0)),
        pl.BlockSpec((RB, Dd), lambda i: (i, 0)),
        pl.BlockSpec((H, H), lambda i: (0, 0)),
        pl.BlockSpec((H,), lambda i: (0,)),
        pl.BlockSpec((Dd, H), lambda i: (0, 0)),
    ]
    args = [part, cnt, xd, wl, bl, wr]
    if head:
        in_specs.append(pl.BlockSpec((H, 1), lambda i: (0, 0)))
        args.append(wh)
    out_specs = [pl.BlockSpec((RB, H), lambda i: (i, 0))]
    out_shape = [jax.ShapeDtypeStruct((NPAD, H), jnp.float32)]
    if emit_chunks:
        out_specs += [pl.BlockSpec((RB, CW), lambda i: (i, 0))] * (H // CW)
        out_shape += [jax.ShapeDtypeStruct((NPAD, CW), jnp.float32)] * (H // CW)
    if head:
        out_specs.append(pl.BlockSpec((RB, 1), lambda i: (i, 0)))
        out_shape.append(jax.ShapeDtypeStruct((NPAD, 1), jnp.float32))
    outs = pl.pallas_call(
        functools.partial(_dense_body, K, emit_chunks, head),
        grid=(NPAD // RB,),
        in_specs=in_specs,
        out_specs=out_specs,
        out_shape=out_shape,
    )(*args)
    full = outs[0]
    chunks = list(outs[1:1 + H // CW]) if emit_chunks else None
    hy = outs[-1] if head else None
    return full, chunks, hy


def _head_kernel():
    mesh = plsc.VectorSubcoreMesh(core_axis_name="c", subcore_axis_name="s")
    PW = MPAD // NW
    scratch = dict(
        yu_v=pltpu.VMEM((NPAD,), jnp.float32),
        ys_v=pltpu.VMEM((NPAD,), jnp.float32),
        mu_v=pltpu.VMEM((PW,), jnp.int32),
        ms_v=pltpu.VMEM((PW,), jnp.int32),
        b_v=pltpu.VMEM((16,), jnp.float32),
        out_v=pltpu.VMEM((PW,), jnp.float32),
    )

    def body(yu_h, ys_h, mu_h, ms_h, b_h, out_h, *, yu_v, ys_v, mu_v, ms_v,
             b_v, out_v):
        c = lax.axis_index("c")
        s = lax.axis_index("s")
        w = s * NC + c
        base = w * PW
        pltpu.sync_copy(yu_h, yu_v)
        pltpu.sync_copy(ys_h, ys_v)
        pltpu.sync_copy(mu_h.at[pl.ds(base, PW)], mu_v)
        pltpu.sync_copy(ms_h.at[pl.ds(base, PW)], ms_v)
        pltpu.sync_copy(b_h, b_v)
        bvec = b_v[...]

        def step(i, carry):
            iu = mu_v[pl.ds(i * 16, 16)]
            is_ = ms_v[pl.ds(i * 16, 16)]
            u = plsc.load_gather(yu_v, [iu])
            v = plsc.load_gather(ys_v, [is_])
            z = u + v + bvec
            out_v[pl.ds(i * 16, 16)] = 1.0 / (1.0 + jnp.exp(-z))
            return carry

        lax.fori_loop(0, PW // 16, step, 0)
        pltpu.sync_copy(out_v, out_h.at[pl.ds(base, PW)])

    return pl.kernel(
        body,
        out_type=jax.ShapeDtypeStruct((MPAD,), jnp.float32),
        mesh=mesh,
        scratch_types=scratch,
        compiler_params=pltpu.CompilerParams(use_tc_tiling_on_sc=False,
                                             needs_layout_passes=False),
    )


def _pad_rows(x, npad):
    return jnp.pad(x, ((0, npad - x.shape[0]),) + ((0, 0),) * (x.ndim - 1))


def kernel(x_user, x_seller, x1, edge_ub, edge_bu, mask,
           emb_age, emb_gender,
           Wl_us1, bl_us1, Wr_us1, Wl_su1, bl_su1, Wr_su1,
           Wl_us2, bl_us2, Wr_us2, Wl_su2, bl_su2, Wr_su2,
           W_lin, b_lin):
    zf = jnp.zeros((RC, CW), jnp.float32)
    zf8 = jnp.zeros((RPT, 8), jnp.float32)
    ones8 = jnp.ones((B, 8), jnp.float32)
    consts = (zf, zf8, ones8)

    def pad_edges(ei):
        srcp = jnp.concatenate(
            [ei[0], jnp.zeros((E2 - E,), ei.dtype)]).reshape(EROWS, B)
        dstp = jnp.concatenate(
            [ei[1], jnp.full((E2 - E,), DUMP, ei.dtype)]).reshape(EROWS, B)
        return srcp, dstp

    x1p = _pad_rows(x1, NPAD)
    xup = _pad_rows(x_user, NPAD)
    xsp = _pad_rows(x_seller, NPAD)

    src_ub, dst_ub = pad_edges(edge_ub)
    src_bu, dst_bu = pad_edges(edge_bu)

    xu_full, xu_chunks, _ = (lambda o: (o[0], o[1], None))(_embed(
        x1p, xup, emb_age, emb_gender))

    xs_chunks = [xsp[:, 0:32], xsp[:, 32:64]]

    def chain(tok, c):
        zf_, zf8_, ones8_ = c
        t = tok * 1e-30
        return (zf_ + t, zf8_ + t, ones8_)

    cnt_ub, cnt_bu = _cnt2(dst_ub, dst_bu, consts)
    part_us1 = _segsum([(xu_chunks, src_ub, dst_ub)],
                       chain(cnt_bu[0, 0, 0], consts))[0]
    part_su1 = _segsum([(xs_chunks, src_bu, dst_bu)],
                       chain(part_us1[0, 0, 0, 0], consts))[0]

    xs1_full, xs1_chunks, _ = _dense(part_us1, cnt_ub, xsp, Wl_us1, bl_us1,
                                     Wr_us1, True)
    xu1_full, xu1_chunks, _ = _dense(part_su1, cnt_bu, xu_full, Wl_su1,
                                     bl_su1, Wr_su1, True)

    part_us2 = _segsum([(xu1_chunks, src_ub, dst_ub)],
                       chain(part_su1[0, 0, 0, 0], consts))[0]
    part_su2 = _segsum([(xs1_chunks, src_bu, dst_bu)],
                       chain(part_us2[0, 0, 0, 0], consts))[0]

    w_top = W_lin[0:128, :]
    w_bot = W_lin[128:256, :]
    _, _, ys = _dense(part_us2, cnt_ub, xs1_full, Wl_us2, bl_us2, Wr_us2,
                      False, wh=w_bot)
    _, _, yu = _dense(part_su2, cnt_bu, xu1_full, Wl_su2, bl_su2, Wr_su2,
                      False, wh=w_top)

    mu = jnp.pad(mask[:, 0], (0, MPAD - M_REAL))
    ms = jnp.pad(mask[:, 1], (0, MPAD - M_REAL))
    b16 = jnp.full((16,), b_lin[0], jnp.float32)

    out = _head_kernel()(yu.reshape(-1), ys.reshape(-1), mu, ms, b16)
    return out[:M_REAL]


M_REAL = 50000
